# Initial kernel scaffold; baseline (speedup 1.0000x reference)
import jax
import jax.numpy as jnp
from jax.experimental import pallas as pl


def kernel(pts3D, feature_ref, feature_map_query, feature_grad_x, feature_grad_y, K):
    raise NotImplementedError("write your pallas kernel here")


# speedup vs baseline: 2.0548x; 2.0548x over previous
import functools

import jax
import jax.numpy as jnp
from jax import lax
from jax.experimental import pallas as pl
from jax.experimental.pallas import tpu as pltpu
from jax.experimental.pallas import tpu_sc as plsc

NW = 32


@functools.lru_cache(maxsize=None)
def _make_prep(C, S, SB, CQ):
    def body(q_ref, gx_ref, gy_ref, tq_ref, tg_ref):
        tq_ref[:, :C] = q_ref[...].T
        tq_ref[:, C:] = jnp.zeros((SB, CQ - C), jnp.float32)
        tg_ref[:, :C] = gx_ref[...].T
        tg_ref[:, C:] = gy_ref[...].T

    grid = (S // SB,)
    return pl.pallas_call(
        body,
        grid=grid,
        in_specs=[
            pl.BlockSpec((C, SB), lambda s: (0, s)),
            pl.BlockSpec((C, SB), lambda s: (0, s)),
            pl.BlockSpec((C, SB), lambda s: (0, s)),
        ],
        out_specs=[
            pl.BlockSpec((SB, CQ), lambda s: (s, 0)),
            pl.BlockSpec((SB, 2 * C), lambda s: (s, 0)),
        ],
        out_shape=[
            jax.ShapeDtypeStruct((S, CQ), jnp.float32),
            jax.ShapeDtypeStruct((S, 2 * C), jnp.float32),
        ],
    )


@functools.lru_cache(maxsize=None)
def _make_project(N, H, W, sub):
    def body(pts_ref, pose_ref, idx_ref):
        px = pts_ref[0, :]
        py = pts_ref[1, :]
        pz = pts_ref[2, :]
        r00, r01, r02 = pose_ref[0], pose_ref[1], pose_ref[2]
        r10, r11, r12 = pose_ref[3], pose_ref[4], pose_ref[5]
        r20, r21, r22 = pose_ref[6], pose_ref[7], pose_ref[8]
        t0, t1, t2 = pose_ref[9], pose_ref[10], pose_ref[11]
        k00, k01, k02 = pose_ref[12], pose_ref[13], pose_ref[14]
        k10, k11, k12 = pose_ref[15], pose_ref[16], pose_ref[17]
        k20, k21, k22 = pose_ref[18], pose_ref[19], pose_ref[20]
        x = px * r00 + py * r01 + pz * r02 + t0
        y = px * r10 + py * r11 + pz * r12 + t1
        z = px * r20 + py * r21 + pz * r22 + t2
        h0 = x * k00 + y * k10 + z * k20
        h1 = x * k01 + y * k11 + z * k21
        h2 = x * k02 + y * k12 + z * k22
        u = h0 / h2
        v = h1 / h2
        iu = u.astype(jnp.int32) - sub
        iv = v.astype(jnp.int32) - sub
        i = jnp.clip(iu, 0, H - 1)
        j = jnp.clip(iv, 0, W - 1)
        idx_ref[...] = i * W + j

    return pl.pallas_call(
        body,
        in_specs=[
            pl.BlockSpec(memory_space=pltpu.VMEM),
            pl.BlockSpec(memory_space=pltpu.SMEM),
        ],
        out_shape=jax.ShapeDtypeStruct((N,), jnp.int32),
    )


@functools.lru_cache(maxsize=None)
def _make_gather_qg(N, C, S, CQ):
    BPW = N // NW
    mesh = plsc.VectorSubcoreMesh(core_axis_name="c", subcore_axis_name="s")
    info = plsc.get_sparse_core_info()
    NC = info.num_cores

    @functools.partial(
        pl.kernel,
        mesh=mesh,
        out_type=[
            jax.ShapeDtypeStruct((N, CQ), jnp.float32),
            jax.ShapeDtypeStruct((N, 2 * C), jnp.float32),
        ],
        scratch_types=[
            pltpu.VMEM((BPW,), jnp.int32),
            pltpu.VMEM((BPW, CQ), jnp.float32),
            pltpu.VMEM((BPW, 2 * C), jnp.float32),
            pltpu.SemaphoreType.DMA,
            pltpu.SemaphoreType.DMA,
        ],
    )
    def k(tq_hbm, tg_hbm, idx_hbm, outq_hbm, outg_hbm, idx_v, q_v, g_v, sem1, sem2):
        wid = lax.axis_index("s") * NC + lax.axis_index("c")
        base = wid * BPW
        pltpu.sync_copy(idx_hbm.at[pl.ds(base, BPW)], idx_v)
        cq = pltpu.async_copy(tq_hbm.at[idx_v], q_v, sem1)
        cg = pltpu.async_copy(tg_hbm.at[idx_v], g_v, sem2)
        cq.wait()
        cg.wait()
        pltpu.sync_copy(q_v, outq_hbm.at[pl.ds(base, BPW)])
        pltpu.sync_copy(g_v, outg_hbm.at[pl.ds(base, BPW)])

    return k


@functools.lru_cache(maxsize=None)
def _make_gather_q(N, C, S, CQ):
    BPW = N // NW
    mesh = plsc.VectorSubcoreMesh(core_axis_name="c", subcore_axis_name="s")
    info = plsc.get_sparse_core_info()
    NC = info.num_cores

    @functools.partial(
        pl.kernel,
        mesh=mesh,
        out_type=jax.ShapeDtypeStruct((N, CQ), jnp.float32),
        scratch_types=[
            pltpu.VMEM((BPW,), jnp.int32),
            pltpu.VMEM((BPW, CQ), jnp.float32),
            pltpu.SemaphoreType.DMA,
        ],
    )
    def k(tq_hbm, idx_hbm, outq_hbm, idx_v, q_v, sem1):
        wid = lax.axis_index("s") * NC + lax.axis_index("c")
        base = wid * BPW
        pltpu.sync_copy(idx_hbm.at[pl.ds(base, BPW)], idx_v)
        pltpu.async_copy(tq_hbm.at[idx_v], q_v, sem1).wait()
        pltpu.sync_copy(q_v, outq_hbm.at[pl.ds(base, BPW)])

    return k


@functools.lru_cache(maxsize=None)
def _make_reduce1(N, C):
    def body(gq_ref, gg_ref, fr_ref, pts_ref, pose_ref, out_ref):
        q = gq_ref[:, :C]
        gx = gg_ref[:, :C]
        gy = gg_ref[:, C:]
        f = fr_ref[...]
        err = q - f
        sgx = jnp.sum(err * gx, axis=-1)
        sgy = jnp.sum(err * gy, axis=-1)
        wxx = jnp.sum(gx * gx, axis=-1)
        wxy = jnp.sum(gx * gy, axis=-1)
        wyy = jnp.sum(gy * gy, axis=-1)
        ee = jnp.sum(err * err, axis=-1)
        px = pts_ref[0, :]
        py = pts_ref[1, :]
        pz = pts_ref[2, :]
        r00, r01, r02 = pose_ref[0], pose_ref[1], pose_ref[2]
        r10, r11, r12 = pose_ref[3], pose_ref[4], pose_ref[5]
        r20, r21, r22 = pose_ref[6], pose_ref[7], pose_ref[8]
        t0, t1, t2 = pose_ref[9], pose_ref[10], pose_ref[11]
        x = px * r00 + py * r01 + pz * r02 + t0
        y = px * r10 + py * r11 + pz * r12 + t1
        z = px * r20 + py * r21 + pz * r22 + t2
        iz = 1.0 / z
        izz = iz * iz
        zero = jnp.zeros_like(x)
        one = jnp.ones_like(x)
        a00, a01, a02 = iz, zero, -x * izz
        a03, a04, a05 = -x * y * izz, 1.0 + x * x * izz, -y * iz
        a10, a11, a12 = zero, iz, -y * izz
        a13, a14, a15 = -1.0 - y * y * izz, x * y * izz, x * iz
        A0T = jnp.stack([a00, a01, a02, a03, a04, a05, zero, ee], axis=0)
        A1T = jnp.stack([a10, a11, a12, a13, a14, a15, zero, zero], axis=0)
        UT = jnp.stack([
            wxx * a00 + wxy * a10, wxx * a01 + wxy * a11,
            wxx * a02 + wxy * a12, wxx * a03 + wxy * a13,
            wxx * a04 + wxy * a14, wxx * a05 + wxy * a15,
            sgx, one,
        ], axis=0)
        VT = jnp.stack([
            wxy * a00 + wyy * a10, wxy * a01 + wyy * a11,
            wxy * a02 + wyy * a12, wxy * a03 + wyy * a13,
            wxy * a04 + wyy * a14, wxy * a05 + wyy * a15,
            sgy, zero,
        ], axis=0)
        dn = (((1,), (1,)), ((), ()))
        out_ref[...] = (
            lax.dot_general(A0T, UT, dn, preferred_element_type=jnp.float32)
            + lax.dot_general(A1T, VT, dn, preferred_element_type=jnp.float32)
        )

    return pl.pallas_call(
        body,
        in_specs=[
            pl.BlockSpec(memory_space=pltpu.VMEM),
            pl.BlockSpec(memory_space=pltpu.VMEM),
            pl.BlockSpec(memory_space=pltpu.VMEM),
            pl.BlockSpec(memory_space=pltpu.VMEM),
            pl.BlockSpec(memory_space=pltpu.SMEM),
        ],
        out_shape=jax.ShapeDtypeStruct((8, 8), jnp.float32),
    )


@functools.lru_cache(maxsize=None)
def _make_reduce2(N, C):
    def body(gq_ref, fr_ref, out_ref):
        err = gq_ref[:, :C] - fr_ref[...]
        out_ref[0, 0] = jnp.sum(err * err)

    return pl.pallas_call(
        body,
        out_specs=pl.BlockSpec(memory_space=pltpu.SMEM),
        out_shape=jax.ShapeDtypeStruct((1, 1), jnp.float32),
    )


def _skew(v):
    z = jnp.zeros_like(v[..., 0])
    M = jnp.stack([z, -v[..., 2], v[..., 1],
                   v[..., 2], z, -v[..., 0],
                   -v[..., 1], v[..., 0], z], axis=-1)
    return M.reshape(v.shape[:-1] + (3, 3))


def _so3exp(w):
    theta = jnp.linalg.norm(w)
    small = theta < 1e-7
    ts = jnp.where(small, 1.0, theta)
    Wm = _skew(w)
    I = jnp.eye(3, dtype=w.dtype)
    R = I + jnp.sin(ts) / ts * Wm + (1.0 - jnp.cos(ts)) / (ts * ts) * (Wm @ Wm)
    return jnp.where(small, I + Wm, R)


def _lm_step(g, H, lambda_):
    D = jnp.diag(jnp.diagonal(H) + 1e-09)
    H = H + D * lambda_
    P = jnp.linalg.inv(H)
    return -(P @ g[..., None])[..., 0]


def kernel(pts3D, feature_ref, feature_map_query, feature_grad_x,
           feature_grad_y, K):
    N, C = feature_ref.shape
    _, H, W = feature_map_query.shape
    S = H * W
    SB = 512
    CQ = ((C + 127) // 128) * 128

    prep = _make_prep(C, S, SB, CQ)
    Tq, Tg = prep(feature_map_query.reshape(C, S),
                  feature_grad_x.reshape(C, S),
                  feature_grad_y.reshape(C, S))

    project1 = _make_project(N, H, W, 1)
    project0 = _make_project(N, H, W, 0)
    gather_qg = _make_gather_qg(N, C, S, CQ)
    gather_q = _make_gather_q(N, C, S, CQ)
    reduce1 = _make_reduce1(N, C)
    reduce2 = _make_reduce2(N, C)

    ptsT = jnp.zeros((8, N), jnp.float32).at[:3, :].set(pts3D.T)

    R = jnp.eye(3, dtype=jnp.float32)
    t = jnp.array([1.0, 1.0, 0.0], dtype=jnp.float32)
    lam = jnp.asarray(0.01, dtype=jnp.float32)
    Kf = K.reshape(-1)
    prev_cost = None

    for it in range(3):
        pose = jnp.concatenate([R.reshape(-1), t, Kf,
                                jnp.zeros((11,), jnp.float32)])
        idx1 = project1(ptsT, pose)
        Gq, Gg = gather_qg(Tq, Tg, idx1)
        out8 = reduce1(Gq, Gg, feature_ref, ptsT, pose)
        Hess = out8[:6, :6]
        Grad = out8[:6, 6]
        if it == 0:
            prev_cost = 0.5 * out8[7, 7] / N
        delta = _lm_step(Grad, Hess, lam)
        dt, dw = delta[:3], delta[3:6]
        dr = _so3exp(dw)
        R_new = dr @ R
        t_new = dr @ t + dt
        pose_new = jnp.concatenate([R_new.reshape(-1), t_new, Kf,
                                    jnp.zeros((11,), jnp.float32)])
        idx2 = project0(ptsT, pose_new)
        Gq2 = gather_q(Tq, idx2)
        new_cost = reduce2(Gq2, feature_ref)[0, 0] / N
        increased = new_cost > prev_cost
        lam = jnp.clip(lam * jnp.where(increased, 10.0, 0.1), 1e-06, 100.0)
        accept = jnp.logical_not(increased)
        prev_cost = jnp.where(accept, new_cost, prev_cost)
        R = jnp.where(accept, R_new, R)
        t = jnp.where(accept, t_new, t)
    return R, t


# speedup vs baseline: 3.7646x; 1.8321x over previous
import functools

import jax
import jax.numpy as jnp
from jax import lax
from jax.experimental import pallas as pl
from jax.experimental.pallas import tpu as pltpu
from jax.experimental.pallas import tpu_sc as plsc

NW = 32


@functools.lru_cache(maxsize=None)
def _make_prep(C, H, W, HB, CQ):
    S = H * W

    def body(q_ref, gx_ref, gy_ref, tq_ref, tg_ref):
        for h in range(HB):
            tq_ref[h * W:(h + 1) * W, :C] = q_ref[:, h, :].T
            tq_ref[h * W:(h + 1) * W, C:] = jnp.zeros((W, CQ - C), jnp.float32)
            tg_ref[h * W:(h + 1) * W, :C] = gx_ref[:, h, :].T
            tg_ref[h * W:(h + 1) * W, C:] = gy_ref[:, h, :].T

    grid = (H // HB,)
    return pl.pallas_call(
        body,
        grid=grid,
        in_specs=[
            pl.BlockSpec((C, HB, W), lambda i: (0, i, 0)),
            pl.BlockSpec((C, HB, W), lambda i: (0, i, 0)),
            pl.BlockSpec((C, HB, W), lambda i: (0, i, 0)),
        ],
        out_specs=[
            pl.BlockSpec((HB * W, CQ), lambda i: (i, 0)),
            pl.BlockSpec((HB * W, 2 * C), lambda i: (i, 0)),
        ],
        out_shape=[
            jax.ShapeDtypeStruct((S, CQ), jnp.float32),
            jax.ShapeDtypeStruct((S, 2 * C), jnp.float32),
        ],
    )


@functools.lru_cache(maxsize=None)
def _make_project(N, H, W, sub):
    def body(pts_ref, pose_ref, idx_ref):
        px = pts_ref[0, :]
        py = pts_ref[1, :]
        pz = pts_ref[2, :]
        r00, r01, r02 = pose_ref[0], pose_ref[1], pose_ref[2]
        r10, r11, r12 = pose_ref[3], pose_ref[4], pose_ref[5]
        r20, r21, r22 = pose_ref[6], pose_ref[7], pose_ref[8]
        t0, t1, t2 = pose_ref[9], pose_ref[10], pose_ref[11]
        k00, k01, k02 = pose_ref[12], pose_ref[13], pose_ref[14]
        k10, k11, k12 = pose_ref[15], pose_ref[16], pose_ref[17]
        k20, k21, k22 = pose_ref[18], pose_ref[19], pose_ref[20]
        x = px * r00 + py * r01 + pz * r02 + t0
        y = px * r10 + py * r11 + pz * r12 + t1
        z = px * r20 + py * r21 + pz * r22 + t2
        h0 = x * k00 + y * k10 + z * k20
        h1 = x * k01 + y * k11 + z * k21
        h2 = x * k02 + y * k12 + z * k22
        u = h0 / h2
        v = h1 / h2
        iu = u.astype(jnp.int32) - sub
        iv = v.astype(jnp.int32) - sub
        i = jnp.clip(iu, 0, H - 1)
        j = jnp.clip(iv, 0, W - 1)
        idx_ref[...] = i * W + j

    return pl.pallas_call(
        body,
        in_specs=[
            pl.BlockSpec(memory_space=pltpu.VMEM),
            pl.BlockSpec(memory_space=pltpu.SMEM),
        ],
        out_shape=jax.ShapeDtypeStruct((N,), jnp.int32),
    )


@functools.lru_cache(maxsize=None)
def _make_gather_qg(N, C, S, CQ):
    BPW = N // NW
    mesh = plsc.VectorSubcoreMesh(core_axis_name="c", subcore_axis_name="s")
    info = plsc.get_sparse_core_info()
    NC = info.num_cores

    @functools.partial(
        pl.kernel,
        mesh=mesh,
        out_type=[
            jax.ShapeDtypeStruct((N, CQ), jnp.float32),
            jax.ShapeDtypeStruct((N, 2 * C), jnp.float32),
        ],
        scratch_types=[
            pltpu.VMEM((BPW,), jnp.int32),
            pltpu.VMEM((BPW, CQ), jnp.float32),
            pltpu.VMEM((BPW, 2 * C), jnp.float32),
            pltpu.SemaphoreType.DMA,
            pltpu.SemaphoreType.DMA,
        ],
    )
    def k(tq_hbm, tg_hbm, idx_hbm, outq_hbm, outg_hbm, idx_v, q_v, g_v, sem1, sem2):
        wid = lax.axis_index("s") * NC + lax.axis_index("c")
        base = wid * BPW
        pltpu.sync_copy(idx_hbm.at[pl.ds(base, BPW)], idx_v)
        cq = pltpu.async_copy(tq_hbm.at[idx_v], q_v, sem1)
        cg = pltpu.async_copy(tg_hbm.at[idx_v], g_v, sem2)
        cq.wait()
        cg.wait()
        pltpu.sync_copy(q_v, outq_hbm.at[pl.ds(base, BPW)])
        pltpu.sync_copy(g_v, outg_hbm.at[pl.ds(base, BPW)])

    return k


@functools.lru_cache(maxsize=None)
def _make_gather_q(N, C, S, CQ):
    BPW = N // NW
    mesh = plsc.VectorSubcoreMesh(core_axis_name="c", subcore_axis_name="s")
    info = plsc.get_sparse_core_info()
    NC = info.num_cores

    @functools.partial(
        pl.kernel,
        mesh=mesh,
        out_type=jax.ShapeDtypeStruct((N, CQ), jnp.float32),
        scratch_types=[
            pltpu.VMEM((BPW,), jnp.int32),
            pltpu.VMEM((BPW, CQ), jnp.float32),
            pltpu.SemaphoreType.DMA,
        ],
    )
    def k(tq_hbm, idx_hbm, outq_hbm, idx_v, q_v, sem1):
        wid = lax.axis_index("s") * NC + lax.axis_index("c")
        base = wid * BPW
        pltpu.sync_copy(idx_hbm.at[pl.ds(base, BPW)], idx_v)
        pltpu.async_copy(tq_hbm.at[idx_v], q_v, sem1).wait()
        pltpu.sync_copy(q_v, outq_hbm.at[pl.ds(base, BPW)])

    return k


@functools.lru_cache(maxsize=None)
def _make_reduce1(N, C):
    def body(gq_ref, gg_ref, fr_ref, pts_ref, pose_ref, out_ref):
        q = gq_ref[:, :C]
        gx = gg_ref[:, :C]
        gy = gg_ref[:, C:]
        f = fr_ref[...]
        err = q - f
        sgx = jnp.sum(err * gx, axis=-1)
        sgy = jnp.sum(err * gy, axis=-1)
        wxx = jnp.sum(gx * gx, axis=-1)
        wxy = jnp.sum(gx * gy, axis=-1)
        wyy = jnp.sum(gy * gy, axis=-1)
        ee = jnp.sum(err * err, axis=-1)
        px = pts_ref[0, :]
        py = pts_ref[1, :]
        pz = pts_ref[2, :]
        r00, r01, r02 = pose_ref[0], pose_ref[1], pose_ref[2]
        r10, r11, r12 = pose_ref[3], pose_ref[4], pose_ref[5]
        r20, r21, r22 = pose_ref[6], pose_ref[7], pose_ref[8]
        t0, t1, t2 = pose_ref[9], pose_ref[10], pose_ref[11]
        x = px * r00 + py * r01 + pz * r02 + t0
        y = px * r10 + py * r11 + pz * r12 + t1
        z = px * r20 + py * r21 + pz * r22 + t2
        iz = 1.0 / z
        izz = iz * iz
        zero = jnp.zeros_like(x)
        one = jnp.ones_like(x)
        a00, a01, a02 = iz, zero, -x * izz
        a03, a04, a05 = -x * y * izz, 1.0 + x * x * izz, -y * iz
        a10, a11, a12 = zero, iz, -y * izz
        a13, a14, a15 = -1.0 - y * y * izz, x * y * izz, x * iz
        A0T = jnp.stack([a00, a01, a02, a03, a04, a05, zero, ee], axis=0)
        A1T = jnp.stack([a10, a11, a12, a13, a14, a15, zero, zero], axis=0)
        UT = jnp.stack([
            wxx * a00 + wxy * a10, wxx * a01 + wxy * a11,
            wxx * a02 + wxy * a12, wxx * a03 + wxy * a13,
            wxx * a04 + wxy * a14, wxx * a05 + wxy * a15,
            sgx, one,
        ], axis=0)
        VT = jnp.stack([
            wxy * a00 + wyy * a10, wxy * a01 + wyy * a11,
            wxy * a02 + wyy * a12, wxy * a03 + wyy * a13,
            wxy * a04 + wyy * a14, wxy * a05 + wyy * a15,
            sgy, zero,
        ], axis=0)
        dn = (((1,), (1,)), ((), ()))
        out_ref[...] = (
            lax.dot_general(A0T, UT, dn, preferred_element_type=jnp.float32)
            + lax.dot_general(A1T, VT, dn, preferred_element_type=jnp.float32)
        )

    return pl.pallas_call(
        body,
        in_specs=[
            pl.BlockSpec(memory_space=pltpu.VMEM),
            pl.BlockSpec(memory_space=pltpu.VMEM),
            pl.BlockSpec(memory_space=pltpu.VMEM),
            pl.BlockSpec(memory_space=pltpu.VMEM),
            pl.BlockSpec(memory_space=pltpu.SMEM),
        ],
        out_shape=jax.ShapeDtypeStruct((8, 8), jnp.float32),
    )


@functools.lru_cache(maxsize=None)
def _make_reduce2(N, C):
    def body(gq_ref, fr_ref, out_ref):
        err = gq_ref[:, :C] - fr_ref[...]
        out_ref[0, 0] = jnp.sum(err * err)

    return pl.pallas_call(
        body,
        out_specs=pl.BlockSpec(memory_space=pltpu.SMEM),
        out_shape=jax.ShapeDtypeStruct((1, 1), jnp.float32),
    )


def _skew(v):
    z = jnp.zeros_like(v[..., 0])
    M = jnp.stack([z, -v[..., 2], v[..., 1],
                   v[..., 2], z, -v[..., 0],
                   -v[..., 1], v[..., 0], z], axis=-1)
    return M.reshape(v.shape[:-1] + (3, 3))


def _so3exp(w):
    theta = jnp.linalg.norm(w)
    small = theta < 1e-7
    ts = jnp.where(small, 1.0, theta)
    Wm = _skew(w)
    I = jnp.eye(3, dtype=w.dtype)
    R = I + jnp.sin(ts) / ts * Wm + (1.0 - jnp.cos(ts)) / (ts * ts) * (Wm @ Wm)
    return jnp.where(small, I + Wm, R)


def _lm_step(g, H, lambda_):
    D = jnp.diag(jnp.diagonal(H) + 1e-09)
    H = H + D * lambda_
    P = jnp.linalg.inv(H)
    return -(P @ g[..., None])[..., 0]


def kernel(pts3D, feature_ref, feature_map_query, feature_grad_x,
           feature_grad_y, K):
    N, C = feature_ref.shape
    _, H, W = feature_map_query.shape
    S = H * W
    CQ = ((C + 127) // 128) * 128

    prep = _make_prep(C, H, W, 8, CQ)
    Tq, Tg = prep(feature_map_query, feature_grad_x, feature_grad_y)

    project1 = _make_project(N, H, W, 1)
    project0 = _make_project(N, H, W, 0)
    gather_qg = _make_gather_qg(N, C, S, CQ)
    gather_q = _make_gather_q(N, C, S, CQ)
    reduce1 = _make_reduce1(N, C)
    reduce2 = _make_reduce2(N, C)

    ptsT = jnp.zeros((8, N), jnp.float32).at[:3, :].set(pts3D.T)

    R = jnp.eye(3, dtype=jnp.float32)
    t = jnp.array([1.0, 1.0, 0.0], dtype=jnp.float32)
    lam = jnp.asarray(0.01, dtype=jnp.float32)
    Kf = K.reshape(-1)
    prev_cost = None

    for it in range(3):
        pose = jnp.concatenate([R.reshape(-1), t, Kf,
                                jnp.zeros((11,), jnp.float32)])
        idx1 = project1(ptsT, pose)
        Gq, Gg = gather_qg(Tq, Tg, idx1)
        out8 = reduce1(Gq, Gg, feature_ref, ptsT, pose)
        Hess = out8[:6, :6]
        Grad = out8[:6, 6]
        if it == 0:
            prev_cost = 0.5 * out8[7, 7] / N
        delta = _lm_step(Grad, Hess, lam)
        dt, dw = delta[:3], delta[3:6]
        dr = _so3exp(dw)
        R_new = dr @ R
        t_new = dr @ t + dt
        pose_new = jnp.concatenate([R_new.reshape(-1), t_new, Kf,
                                    jnp.zeros((11,), jnp.float32)])
        idx2 = project0(ptsT, pose_new)
        Gq2 = gather_q(Tq, idx2)
        new_cost = reduce2(Gq2, feature_ref)[0, 0] / N
        increased = new_cost > prev_cost
        lam = jnp.clip(lam * jnp.where(increased, 10.0, 0.1), 1e-06, 100.0)
        accept = jnp.logical_not(increased)
        prev_cost = jnp.where(accept, new_cost, prev_cost)
        R = jnp.where(accept, R_new, R)
        t = jnp.where(accept, t_new, t)
    return R, t
